# baseline (speedup 1.0000x reference)
import jax
import jax.numpy as jnp
from jax import lax
from jax.experimental import pallas as pl
from jax.experimental.pallas import tpu as pltpu
from jax.experimental.pallas import tpu_sc as plsc

_NC = 2
_NS = 16
_NW = _NC * _NS
_T = 8192
_CHUNK = _T // _NW
_RG = 2048
_R = 512


def _quant_body(w1_ref, w0_ref, w1q_ref, w0b_ref):
    w = w1_ref[...]
    bits = lax.bitcast_convert_type(jnp.abs(w), jnp.int32)
    e = (bits >> 23) - 127
    m = bits & 0x7FFFFF
    shift = e + jnp.where(m >= 0x3504F3, 1, 0)
    shift = jnp.clip(shift, -14, 0)
    pow2 = lax.bitcast_convert_type((shift + 127) << 23, jnp.float32)
    w1q = jnp.sign(w) * pow2
    w1q_ref[...] = w1q.astype(jnp.bfloat16)
    w0b_ref[...] = w0_ref[...].astype(jnp.bfloat16)


def _gate_body(x_ref, wg_ref, bg_ref, eid_ref, xb_ref):
    x = x_ref[...]
    xb = x.astype(jnp.bfloat16)
    logits = lax.dot_general(
        xb, wg_ref[...].astype(jnp.bfloat16),
        (((1,), (0,)), ((), ())),
        preferred_element_type=jnp.float32,
    ) + bg_ref[...]
    lt = jnp.transpose(logits)
    eid = (lt[1:2, :] > lt[0:1, :]).astype(jnp.int32)
    eid_ref[...] = eid.reshape(1, 1, _RG)
    xb_ref[...] = xb


def _splat_total(v):
    r = lax.rev(plsc.cumsum(lax.rev(v, (0,))), (0,))
    return plsc.cumsum(v) + r - v


def _route_body(eid_hbm, xb_hbm, invp_hbm, n0_hbm, xs_hbm,
                eid_v, posflat, pos_rows, n0_v, rb0, rb1,
                gs0, gs1, ws0, ws1):
    w = lax.axis_index("s") * _NC + lax.axis_index("c")
    base = w * _CHUNK

    pltpu.sync_copy(eid_hbm, eid_v)

    def count_step(i, carry):
        accb, acct = carry
        v = eid_v[pl.ds(i * 16, 16)]
        z = 1 - v
        li = i * 16 + lax.iota(jnp.int32, 16)
        accb = accb + jnp.where(li < base, z, 0)
        return accb, acct + z

    accb, acct = lax.fori_loop(
        0, _T // 16, count_step,
        (jnp.zeros((16,), jnp.int32), jnp.zeros((16,), jnp.int32)))
    base0 = _splat_total(accb)
    n0 = _splat_total(acct)
    base1 = n0 + base - base0

    zc = jnp.zeros((16,), jnp.int32)
    for j in range(_CHUNK // 16):
        v = eid_v[pl.ds(base + j * 16, 16)]
        z = 1 - v
        cz = plsc.cumsum(z)
        excl = cz - z
        li = j * 16 + lax.iota(jnp.int32, 16)
        pos = jnp.where(v == 0,
                        base0 + zc + excl,
                        base1 + li - (zc + excl))
        posflat[pl.ds(j * 16, 16)] = pos
        pos_rows[j // 4, pl.ds((j % 4) * 16, 16)] = pos
        zc = zc + _splat_total(z)

    pltpu.sync_copy(posflat, invp_hbm.at[w])

    @pl.when(w == 0)
    def _():
        n0_v[pl.ds(0, 16)] = n0
        pltpu.sync_copy(n0_v, n0_hbm)

    bufs, gsems, wsems = (rb0, rb1), (gs0, gs1), (ws0, ws1)
    gd = [None, None]
    wd = [None, None]
    gd[0] = pltpu.async_copy(
        xb_hbm.at[pl.ds(base, 64)], bufs[0], gsems[0])
    for k in range(4):
        b = k % 2
        nb = (k + 1) % 2
        gd[b].wait()
        if k + 1 < 4:
            if wd[nb] is not None:
                wd[nb].wait()
            gd[nb] = pltpu.async_copy(
                xb_hbm.at[pl.ds(base + (k + 1) * 64, 64)], bufs[nb],
                gsems[nb])
        wd[b] = pltpu.async_copy(bufs[b], xs_hbm.at[pos_rows.at[k]],
                                 wsems[b])
    wd[0].wait()
    wd[1].wait()


def _mm_body(n0_ref, xs_ref, w0_ref, w1_ref, b0_ref, b1_ref, o_ref):
    n0 = n0_ref[0]
    lo = pl.program_id(0) * _R
    xb = xs_ref[...]
    dims = (((1,), (0,)), ((), ()))

    @pl.when(lo + _R <= n0)
    def _():
        o_ref[...] = lax.dot_general(
            xb, w0_ref[...], dims, preferred_element_type=jnp.float32,
        ) + b0_ref[...]

    @pl.when(lo >= n0)
    def _():
        o_ref[...] = lax.dot_general(
            xb, w1_ref[...], dims, preferred_element_type=jnp.float32,
        ) + b1_ref[...]

    @pl.when(jnp.logical_and(lo < n0, lo + _R > n0))
    def _():
        out0 = lax.dot_general(
            xb, w0_ref[...], dims, preferred_element_type=jnp.float32,
        ) + b0_ref[...]
        out1 = lax.dot_general(
            xb, w1_ref[...], dims, preferred_element_type=jnp.float32,
        ) + b1_ref[...]
        rows = lo + lax.broadcasted_iota(jnp.int32, (_R, 1), 0)
        o_ref[...] = jnp.where(rows < n0, out0, out1)


def _unsort_body(ys_hbm, invp_hbm, y_hbm, idx_v, rb0, rb1,
                 gs0, gs1, ws0, ws1):
    w = lax.axis_index("s") * _NC + lax.axis_index("c")
    base = w * _CHUNK
    pltpu.sync_copy(invp_hbm.at[w], idx_v)
    bufs, gsems, wsems = (rb0, rb1), (gs0, gs1), (ws0, ws1)
    nch = _CHUNK // 8
    gd = [None, None]
    wd = [None, None]
    gd[0] = pltpu.async_copy(
        ys_hbm.at[idx_v.at[pl.ds(0, 8)]], bufs[0], gsems[0])
    for k in range(nch):
        b = k % 2
        nb = (k + 1) % 2
        gd[b].wait()
        if k + 1 < nch:
            if wd[nb] is not None:
                wd[nb].wait()
            gd[nb] = pltpu.async_copy(
                ys_hbm.at[idx_v.at[pl.ds((k + 1) * 8, 8)]], bufs[nb],
                gsems[nb])
        wd[b] = pltpu.async_copy(
            bufs[b], y_hbm.at[pl.ds(base + k * 8, 8)], wsems[b])
    wd[(nch - 2) % 2].wait()
    wd[(nch - 1) % 2].wait()


@jax.jit
def _run(x, Wg, bg, W0, b0, W1, b1):
    T, C = x.shape
    H = W0.shape[1]

    w1q, w0b = pl.pallas_call(
        _quant_body,
        grid=(4,),
        in_specs=[
            pl.BlockSpec((C, H // 4), lambda j: (0, j)),
            pl.BlockSpec((C, H // 4), lambda j: (0, j)),
        ],
        out_specs=[
            pl.BlockSpec((C, H // 4), lambda j: (0, j)),
            pl.BlockSpec((C, H // 4), lambda j: (0, j)),
        ],
        out_shape=[
            jax.ShapeDtypeStruct((C, H), jnp.bfloat16),
            jax.ShapeDtypeStruct((C, H), jnp.bfloat16),
        ],
    )(W1, W0)

    eid3, xb = pl.pallas_call(
        _gate_body,
        grid=(T // _RG,),
        in_specs=[
            pl.BlockSpec((_RG, C), lambda i: (i, 0)),
            pl.BlockSpec((C, 2), lambda i: (0, 0)),
            pl.BlockSpec((1, 2), lambda i: (0, 0)),
        ],
        out_specs=[
            pl.BlockSpec((1, 1, _RG), lambda i: (i, 0, 0)),
            pl.BlockSpec((_RG, C), lambda i: (i, 0)),
        ],
        out_shape=[
            jax.ShapeDtypeStruct((T // _RG, 1, _RG), jnp.int32),
            jax.ShapeDtypeStruct((T, C), jnp.bfloat16),
        ],
    )(x, Wg, bg.reshape(1, 2))
    eid = eid3.reshape(T)
    xbi = lax.bitcast_convert_type(xb.reshape(T, C // 2, 2), jnp.int32)

    mesh = plsc.VectorSubcoreMesh(
        core_axis_name="c", subcore_axis_name="s",
        num_cores=_NC, num_subcores=_NS)
    invp, n0a, xsi = pl.kernel(
        _route_body,
        out_type=[
            jax.ShapeDtypeStruct((_NW, _CHUNK), jnp.int32),
            jax.ShapeDtypeStruct((16,), jnp.int32),
            jax.ShapeDtypeStruct((T, C // 2), jnp.int32),
        ],
        mesh=mesh,
        scratch_types=[
            pltpu.VMEM((_T,), jnp.int32),
            pltpu.VMEM((_CHUNK,), jnp.int32),
            pltpu.VMEM((4, 64), jnp.int32),
            pltpu.VMEM((16,), jnp.int32),
            pltpu.VMEM((64, C // 2), jnp.int32),
            pltpu.VMEM((64, C // 2), jnp.int32),
            pltpu.SemaphoreType.DMA,
            pltpu.SemaphoreType.DMA,
            pltpu.SemaphoreType.DMA,
            pltpu.SemaphoreType.DMA,
        ],
        compiler_params=pltpu.CompilerParams(needs_layout_passes=False),
    )(eid, xbi)
    xs = lax.bitcast_convert_type(xsi, jnp.bfloat16).reshape(T, C)

    ys = pl.pallas_call(
        _mm_body,
        grid_spec=pltpu.PrefetchScalarGridSpec(
            num_scalar_prefetch=1,
            grid=(T // _R,),
            in_specs=[
                pl.BlockSpec((_R, C), lambda i, n0: (i, 0)),
                pl.BlockSpec((C, H), lambda i, n0: (0, 0)),
                pl.BlockSpec((C, H), lambda i, n0: (0, 0)),
                pl.BlockSpec((1, H), lambda i, n0: (0, 0)),
                pl.BlockSpec((1, H), lambda i, n0: (0, 0)),
            ],
            out_specs=pl.BlockSpec((_R, H), lambda i, n0: (i, 0)),
        ),
        out_shape=jax.ShapeDtypeStruct((T, H), jnp.float32),
    )(n0a, xs, w0b, w1q, b0.reshape(1, H), b1.reshape(1, H))

    y = pl.kernel(
        _unsort_body,
        out_type=jax.ShapeDtypeStruct((T, H), jnp.float32),
        mesh=mesh,
        scratch_types=[
            pltpu.VMEM((_CHUNK,), jnp.int32),
            pltpu.VMEM((8, H), jnp.float32),
            pltpu.VMEM((8, H), jnp.float32),
            pltpu.SemaphoreType.DMA,
            pltpu.SemaphoreType.DMA,
            pltpu.SemaphoreType.DMA,
            pltpu.SemaphoreType.DMA,
        ],
        compiler_params=pltpu.CompilerParams(needs_layout_passes=False),
    )(ys, invp)
    return y


def kernel(inp, Wg, bg, W0, b0, W1, b1):
    B, N, C = inp.shape
    x = inp.reshape(-1, C)
    y = _run(x, Wg, bg, W0, b0, W1, b1)
    return y.reshape(B, N, -1)


# speedup vs baseline: 2.0880x; 2.0880x over previous
import jax
import jax.numpy as jnp
from jax import lax
from jax.experimental import pallas as pl
from jax.experimental.pallas import tpu as pltpu
from jax.experimental.pallas import tpu_sc as plsc

_NC = 2
_NS = 16
_NW = _NC * _NS
_T = 8192
_CHUNK = _T // _NW
_RG = 2048
_R = 256


def _quant_body(w1_ref, w0_ref, w1q_ref, w0b_ref):
    w = w1_ref[...]
    bits = lax.bitcast_convert_type(jnp.abs(w), jnp.int32)
    e = (bits >> 23) - 127
    m = bits & 0x7FFFFF
    shift = e + jnp.where(m >= 0x3504F3, 1, 0)
    shift = jnp.clip(shift, -14, 0)
    pow2 = lax.bitcast_convert_type((shift + 127) << 23, jnp.float32)
    w1q = jnp.sign(w) * pow2
    w1q_ref[...] = w1q.astype(jnp.bfloat16)
    w0b_ref[...] = w0_ref[...].astype(jnp.bfloat16)


def _gate_body(x_ref, wg_ref, bg_ref, eid_ref):
    x = x_ref[...]
    logits = lax.dot_general(
        x.astype(jnp.bfloat16), wg_ref[...].astype(jnp.bfloat16),
        (((1,), (0,)), ((), ())),
        preferred_element_type=jnp.float32,
    ) + bg_ref[...]
    lt = jnp.transpose(logits)
    eid = (lt[1:2, :] > lt[0:1, :]).astype(jnp.int32)
    eid_ref[...] = eid.reshape(1, 1, _RG)


def _splat_total(v):
    r = lax.rev(plsc.cumsum(lax.rev(v, (0,))), (0,))
    return plsc.cumsum(v) + r - v


def _route_body(eid_hbm, xb_hbm, invp_hbm, n0_hbm, xs_hbm,
                eid_v, posflat, pos_rows2, n0_v, rb0, rb1,
                gs0, gs1, ws0, ws1):
    w = lax.axis_index("s") * _NC + lax.axis_index("c")
    base = w * _CHUNK

    pltpu.sync_copy(eid_hbm, eid_v)

    def count_step(i, carry):
        accb, acct = carry
        v = eid_v[pl.ds(i * 16, 16)]
        z = 1 - v
        li = i * 16 + lax.iota(jnp.int32, 16)
        accb = accb + jnp.where(li < base, z, 0)
        return accb, acct + z

    accb, acct = lax.fori_loop(
        0, _T // 16, count_step,
        (jnp.zeros((16,), jnp.int32), jnp.zeros((16,), jnp.int32)))
    base0 = _splat_total(accb)
    n0 = _splat_total(acct)
    base1 = n0 + base - base0

    zc = jnp.zeros((16,), jnp.int32)
    for j in range(_CHUNK // 16):
        v = eid_v[pl.ds(base + j * 16, 16)]
        z = 1 - v
        cz = plsc.cumsum(z)
        excl = cz - z
        li = j * 16 + lax.iota(jnp.int32, 16)
        pos = jnp.where(v == 0,
                        base0 + zc + excl,
                        base1 + li - (zc + excl))
        posflat[pl.ds(j * 16, 16)] = pos
        pos_rows2[j // 2, pl.ds((j % 2) * 16, 16)] = pos
        zc = zc + _splat_total(z)

    pltpu.sync_copy(posflat, invp_hbm.at[w])

    @pl.when(w == 0)
    def _():
        n0_v[pl.ds(0, 16)] = n0
        pltpu.sync_copy(n0_v, n0_hbm)

    bufs, gsems, wsems = (rb0, rb1), (gs0, gs1), (ws0, ws1)
    nch = _CHUNK // 32
    gd = [None, None]
    wd = [None, None]
    gd[0] = pltpu.async_copy(
        xb_hbm.at[pl.ds(base, 32)], bufs[0], gsems[0])
    for k in range(nch):
        b = k % 2
        nb = (k + 1) % 2
        gd[b].wait()
        if k + 1 < nch:
            if wd[nb] is not None:
                wd[nb].wait()
            gd[nb] = pltpu.async_copy(
                xb_hbm.at[pl.ds(base + (k + 1) * 32, 32)], bufs[nb],
                gsems[nb])
        wd[b] = pltpu.async_copy(
            bufs[b], xs_hbm.at[pos_rows2.at[k]], wsems[b])
    wd[(nch - 2) % 2].wait()
    wd[(nch - 1) % 2].wait()


def _mm_body(n0_ref, xs_ref, w0_ref, w1_ref, b0_ref, b1_ref, o_ref):
    n0 = n0_ref[0]
    lo = pl.program_id(0) * _R
    xb = xs_ref[...].astype(jnp.bfloat16)
    dims = (((1,), (0,)), ((), ()))

    @pl.when(lo + _R <= n0)
    def _():
        o_ref[...] = lax.dot_general(
            xb, w0_ref[...], dims, preferred_element_type=jnp.float32,
        ) + b0_ref[...]

    @pl.when(lo >= n0)
    def _():
        o_ref[...] = lax.dot_general(
            xb, w1_ref[...], dims, preferred_element_type=jnp.float32,
        ) + b1_ref[...]

    @pl.when(jnp.logical_and(lo < n0, lo + _R > n0))
    def _():
        out0 = lax.dot_general(
            xb, w0_ref[...], dims, preferred_element_type=jnp.float32,
        ) + b0_ref[...]
        out1 = lax.dot_general(
            xb, w1_ref[...], dims, preferred_element_type=jnp.float32,
        ) + b1_ref[...]
        rows = lo + lax.broadcasted_iota(jnp.int32, (_R, 1), 0)
        o_ref[...] = jnp.where(rows < n0, out0, out1)


def _unsort_body(ys_hbm, invp_hbm, y_hbm, idx_v, rb0, rb1,
                 gs0, gs1, ws0, ws1):
    w = lax.axis_index("s") * _NC + lax.axis_index("c")
    base = w * _CHUNK
    pltpu.sync_copy(invp_hbm.at[w], idx_v)
    bufs, gsems, wsems = (rb0, rb1), (gs0, gs1), (ws0, ws1)
    nch = _CHUNK // 8
    gd = [None, None]
    wd = [None, None]
    gd[0] = pltpu.async_copy(
        ys_hbm.at[idx_v.at[pl.ds(0, 8)]], bufs[0], gsems[0])
    for k in range(nch):
        b = k % 2
        nb = (k + 1) % 2
        gd[b].wait()
        if k + 1 < nch:
            if wd[nb] is not None:
                wd[nb].wait()
            gd[nb] = pltpu.async_copy(
                ys_hbm.at[idx_v.at[pl.ds((k + 1) * 8, 8)]], bufs[nb],
                gsems[nb])
        wd[b] = pltpu.async_copy(
            bufs[b], y_hbm.at[pl.ds(base + k * 8, 8)], wsems[b])
    wd[(nch - 2) % 2].wait()
    wd[(nch - 1) % 2].wait()


@jax.jit
def _run(x, Wg, bg, W0, b0, W1, b1):
    T, C = x.shape
    H = W0.shape[1]

    w1q, w0b = pl.pallas_call(
        _quant_body,
        grid=(4,),
        in_specs=[
            pl.BlockSpec((C, H // 4), lambda j: (0, j)),
            pl.BlockSpec((C, H // 4), lambda j: (0, j)),
        ],
        out_specs=[
            pl.BlockSpec((C, H // 4), lambda j: (0, j)),
            pl.BlockSpec((C, H // 4), lambda j: (0, j)),
        ],
        out_shape=[
            jax.ShapeDtypeStruct((C, H), jnp.bfloat16),
            jax.ShapeDtypeStruct((C, H), jnp.bfloat16),
        ],
    )(W1, W0)

    eid3 = pl.pallas_call(
        _gate_body,
        grid=(T // _RG,),
        in_specs=[
            pl.BlockSpec((_RG, C), lambda i: (i, 0)),
            pl.BlockSpec((C, 2), lambda i: (0, 0)),
            pl.BlockSpec((1, 2), lambda i: (0, 0)),
        ],
        out_specs=pl.BlockSpec((1, 1, _RG), lambda i: (i, 0, 0)),
        out_shape=jax.ShapeDtypeStruct((T // _RG, 1, _RG), jnp.int32),
    )(x, Wg, bg.reshape(1, 2))
    eid = eid3.reshape(T)

    mesh = plsc.VectorSubcoreMesh(
        core_axis_name="c", subcore_axis_name="s",
        num_cores=_NC, num_subcores=_NS)
    invp, n0a, xs = pl.kernel(
        _route_body,
        out_type=[
            jax.ShapeDtypeStruct((_NW, _CHUNK), jnp.int32),
            jax.ShapeDtypeStruct((16,), jnp.int32),
            jax.ShapeDtypeStruct((T, C), jnp.float32),
        ],
        mesh=mesh,
        scratch_types=[
            pltpu.VMEM((_T,), jnp.int32),
            pltpu.VMEM((_CHUNK,), jnp.int32),
            pltpu.VMEM((8, 32), jnp.int32),
            pltpu.VMEM((16,), jnp.int32),
            pltpu.VMEM((32, C), jnp.float32),
            pltpu.VMEM((32, C), jnp.float32),
            pltpu.SemaphoreType.DMA,
            pltpu.SemaphoreType.DMA,
            pltpu.SemaphoreType.DMA,
            pltpu.SemaphoreType.DMA,
        ],
        compiler_params=pltpu.CompilerParams(needs_layout_passes=False),
    )(eid, x)

    ys = pl.pallas_call(
        _mm_body,
        grid_spec=pltpu.PrefetchScalarGridSpec(
            num_scalar_prefetch=1,
            grid=(T // _R,),
            in_specs=[
                pl.BlockSpec((_R, C), lambda i, n0: (i, 0)),
                pl.BlockSpec((C, H), lambda i, n0: (0, 0)),
                pl.BlockSpec((C, H), lambda i, n0: (0, 0)),
                pl.BlockSpec((1, H), lambda i, n0: (0, 0)),
                pl.BlockSpec((1, H), lambda i, n0: (0, 0)),
            ],
            out_specs=pl.BlockSpec((_R, H), lambda i, n0: (i, 0)),
        ),
        out_shape=jax.ShapeDtypeStruct((T, H), jnp.float32),
    )(n0a, xs, w0b, w1q, b0.reshape(1, H), b1.reshape(1, H))

    y = pl.kernel(
        _unsort_body,
        out_type=jax.ShapeDtypeStruct((T, H), jnp.float32),
        mesh=mesh,
        scratch_types=[
            pltpu.VMEM((_CHUNK,), jnp.int32),
            pltpu.VMEM((8, H), jnp.float32),
            pltpu.VMEM((8, H), jnp.float32),
            pltpu.SemaphoreType.DMA,
            pltpu.SemaphoreType.DMA,
            pltpu.SemaphoreType.DMA,
            pltpu.SemaphoreType.DMA,
        ],
        compiler_params=pltpu.CompilerParams(needs_layout_passes=False),
    )(ys, invp)
    return y


def kernel(inp, Wg, bg, W0, b0, W1, b1):
    B, N, C = inp.shape
    x = inp.reshape(-1, C)
    y = _run(x, Wg, bg, W0, b0, W1, b1)
    return y.reshape(B, N, -1)


# speedup vs baseline: 2.7524x; 1.3182x over previous
import jax
import jax.numpy as jnp
from jax import lax
from jax.experimental import pallas as pl
from jax.experimental.pallas import tpu as pltpu

_RG = 2048
_R = 256


def _quant_body(w1_ref, w0_ref, w1q_ref, w0b_ref):
    w = w1_ref[...]
    bits = lax.bitcast_convert_type(jnp.abs(w), jnp.int32)
    e = (bits >> 23) - 127
    m = bits & 0x7FFFFF
    shift = e + jnp.where(m >= 0x3504F3, 1, 0)
    shift = jnp.clip(shift, -14, 0)
    pow2 = lax.bitcast_convert_type((shift + 127) << 23, jnp.float32)
    w1q = jnp.sign(w) * pow2
    w1q_ref[...] = w1q.astype(jnp.bfloat16)
    w0b_ref[...] = w0_ref[...].astype(jnp.bfloat16)


def _gate_cast_body(x_ref, wg_ref, bg_ref, m_ref, xb_ref):
    x = x_ref[...]
    xb = x.astype(jnp.bfloat16)
    logits = lax.dot_general(
        xb, wg_ref[...].astype(jnp.bfloat16),
        (((1,), (0,)), ((), ())),
        preferred_element_type=jnp.float32,
    ) + bg_ref[...]
    m_ref[...] = (logits[:, 1:2] > logits[:, 0:1]).astype(jnp.float32)
    xb_ref[...] = xb


def _dense_body(xb_ref, m_ref, w0_ref, w1_ref, b0_ref, b1_ref, o_ref):
    xb = xb_ref[...]
    dims = (((1,), (0,)), ((), ()))
    out0 = lax.dot_general(
        xb, w0_ref[...], dims, preferred_element_type=jnp.float32,
    ) + b0_ref[...]
    out1 = lax.dot_general(
        xb, w1_ref[...], dims, preferred_element_type=jnp.float32,
    ) + b1_ref[...]
    o_ref[...] = jnp.where(m_ref[...] > 0.5, out1, out0)


@jax.jit
def _run(x, Wg, bg, W0, b0, W1, b1):
    T, C = x.shape
    H = W0.shape[1]

    w1q, w0b = pl.pallas_call(
        _quant_body,
        grid=(4,),
        in_specs=[
            pl.BlockSpec((C, H // 4), lambda j: (0, j)),
            pl.BlockSpec((C, H // 4), lambda j: (0, j)),
        ],
        out_specs=[
            pl.BlockSpec((C, H // 4), lambda j: (0, j)),
            pl.BlockSpec((C, H // 4), lambda j: (0, j)),
        ],
        out_shape=[
            jax.ShapeDtypeStruct((C, H), jnp.bfloat16),
            jax.ShapeDtypeStruct((C, H), jnp.bfloat16),
        ],
    )(W1, W0)

    m, xb = pl.pallas_call(
        _gate_cast_body,
        grid=(T // _RG,),
        in_specs=[
            pl.BlockSpec((_RG, C), lambda i: (i, 0)),
            pl.BlockSpec((C, 2), lambda i: (0, 0)),
            pl.BlockSpec((1, 2), lambda i: (0, 0)),
        ],
        out_specs=[
            pl.BlockSpec((_RG, 1), lambda i: (i, 0)),
            pl.BlockSpec((_RG, C), lambda i: (i, 0)),
        ],
        out_shape=[
            jax.ShapeDtypeStruct((T, 1), jnp.float32),
            jax.ShapeDtypeStruct((T, C), jnp.bfloat16),
        ],
    )(x, Wg, bg.reshape(1, 2))

    y = pl.pallas_call(
        _dense_body,
        grid=(T // _R,),
        in_specs=[
            pl.BlockSpec((_R, C), lambda i: (i, 0)),
            pl.BlockSpec((_R, 1), lambda i: (i, 0)),
            pl.BlockSpec((C, H), lambda i: (0, 0)),
            pl.BlockSpec((C, H), lambda i: (0, 0)),
            pl.BlockSpec((1, H), lambda i: (0, 0)),
            pl.BlockSpec((1, H), lambda i: (0, 0)),
        ],
        out_specs=pl.BlockSpec((_R, H), lambda i: (i, 0)),
        out_shape=jax.ShapeDtypeStruct((T, H), jnp.float32),
    )(xb, m, w0b, w1q, b0.reshape(1, H), b1.reshape(1, H))
    return y


def kernel(inp, Wg, bg, W0, b0, W1, b1):
    B, N, C = inp.shape
    x = inp.reshape(-1, C)
    y = _run(x, Wg, bg, W0, b0, W1, b1)
    return y.reshape(B, N, -1)


# speedup vs baseline: 3.0684x; 1.1148x over previous
import functools

import jax
import jax.numpy as jnp
from jax.experimental import pallas as pl
from jax.experimental.pallas import tpu as pltpu


def _quant_body(w1_ref, w0_ref, w1q_ref, w0b_ref):
    w = w1_ref[...]
    bits = jax.lax.bitcast_convert_type(jnp.abs(w), jnp.int32)
    e = (bits >> 23) - 127
    m = bits & 0x7FFFFF
    shift = e + jnp.where(m >= 0x3504F3, 1, 0)
    shift = jnp.clip(shift, -14, 0)
    pow2 = jax.lax.bitcast_convert_type((shift + 127) << 23, jnp.float32)
    w1q = jnp.sign(w) * pow2
    w1q_ref[...] = w1q.astype(jnp.bfloat16)
    w0b_ref[...] = w0_ref[...].astype(jnp.bfloat16)


def _moe_body(x_ref, wg_ref, bg_ref, w0_ref, w1_ref, b0_ref, b1_ref, o_ref):
    x = x_ref[...]
    logits = jax.lax.dot_general(
        x.astype(jnp.bfloat16), wg_ref[...].astype(jnp.bfloat16),
        (((1,), (0,)), ((), ())),
        preferred_element_type=jnp.float32,
    ) + bg_ref[...]
    take1 = logits[:, 1:2] > logits[:, 0:1]
    xb = x.astype(jnp.bfloat16)
    out0 = jax.lax.dot_general(
        xb, w0_ref[...], (((1,), (0,)), ((), ())),
        preferred_element_type=jnp.float32,
    ) + b0_ref[...]
    out1 = jax.lax.dot_general(
        xb, w1_ref[...], (((1,), (0,)), ((), ())),
        preferred_element_type=jnp.float32,
    ) + b1_ref[...]
    o_ref[...] = jnp.where(take1, out1, out0)


@functools.partial(jax.jit, static_argnames=("interpret",))
def _run(x, Wg, bg, W0, b0, W1, b1, interpret=False):
    T, C = x.shape
    H = W0.shape[1]
    R = 512

    w1q, w0b = pl.pallas_call(
        _quant_body,
        grid=(4,),
        in_specs=[
            pl.BlockSpec((C, H // 4), lambda j: (0, j)),
            pl.BlockSpec((C, H // 4), lambda j: (0, j)),
        ],
        out_specs=[
            pl.BlockSpec((C, H // 4), lambda j: (0, j)),
            pl.BlockSpec((C, H // 4), lambda j: (0, j)),
        ],
        out_shape=[
            jax.ShapeDtypeStruct((C, H), jnp.bfloat16),
            jax.ShapeDtypeStruct((C, H), jnp.bfloat16),
        ],
        interpret=interpret,
    )(W1, W0)

    y = pl.pallas_call(
        _moe_body,
        grid=(T // R,),
        in_specs=[
            pl.BlockSpec((R, C), lambda i: (i, 0)),
            pl.BlockSpec((C, 2), lambda i: (0, 0)),
            pl.BlockSpec((1, 2), lambda i: (0, 0)),
            pl.BlockSpec((C, H), lambda i: (0, 0)),
            pl.BlockSpec((C, H), lambda i: (0, 0)),
            pl.BlockSpec((1, H), lambda i: (0, 0)),
            pl.BlockSpec((1, H), lambda i: (0, 0)),
        ],
        out_specs=pl.BlockSpec((R, H), lambda i: (i, 0)),
        out_shape=jax.ShapeDtypeStruct((T, H), jnp.float32),
        interpret=interpret,
    )(x, Wg, bg.reshape(1, 2), w0b, w1q, b0.reshape(1, H), b1.reshape(1, H))
    return y


def kernel(inp, Wg, bg, W0, b0, W1, b1):
    B, N, C = inp.shape
    x = inp.reshape(-1, C)
    y = _run(x, Wg, bg, W0, b0, W1, b1)
    return y.reshape(B, N, -1)
